# Initial kernel scaffold; baseline (speedup 1.0000x reference)
import jax
import jax.numpy as jnp
from jax.experimental import pallas as pl


def kernel(input_ids, embedding_table):
    raise NotImplementedError("write your pallas kernel here")


# speedup vs baseline: 2.3060x; 2.3060x over previous
import functools

import jax
import jax.numpy as jnp
from jax import lax
from jax.experimental import pallas as pl
from jax.experimental.pallas import tpu as pltpu
from jax.experimental.pallas import tpu_sc as plsc

VOCAB = 1000000
EMB = 32
B = 4096
L = 200

NC = 2
NS = 16
NW = NC * NS
BPW = B // NW
HALF = L // 2


def _body(ids_hbm, table_hbm, out_hbm, idx_v, buf_a, buf_b, out_v, sem_a, sem_b):
    c = lax.axis_index("c")
    s = lax.axis_index("s")
    wid = s * NC + c
    row0 = wid * BPW

    pltpu.sync_copy(ids_hbm.at[pl.ds(row0 * 2, 2 * BPW)], idx_v)

    def fire(r, buf, sem):
        pltpu.async_copy(table_hbm.at[idx_v.at[2 * r]], buf.at[pl.ds(0, HALF)], sem)
        pltpu.async_copy(table_hbm.at[idx_v.at[2 * r + 1]], buf.at[pl.ds(HALF, HALF)], sem)

    def drain(buf, sem):
        pltpu.make_async_copy(table_hbm.at[pl.ds(0, L)], buf, sem).wait()

    def accum(buf, r):
        zero = jnp.zeros((16,), jnp.float32)

        def body(j, carry):
            a0, a1 = carry
            a0 = a0 + buf[j, pl.ds(0, 16)]
            a1 = a1 + buf[j, pl.ds(16, 16)]
            return a0, a1

        a0, a1 = lax.fori_loop(0, L, body, (zero, zero), unroll=8)
        out_v[r, pl.ds(0, 16)] = a0
        out_v[r, pl.ds(16, 16)] = a1

    fire(0, buf_a, sem_a)
    fire(1, buf_b, sem_b)

    def step(i, _):
        g = 2 * i
        drain(buf_a, sem_a)
        accum(buf_a, g)
        fire(jnp.minimum(g + 2, BPW - 1), buf_a, sem_a)
        drain(buf_b, sem_b)
        accum(buf_b, g + 1)
        fire(jnp.minimum(g + 3, BPW - 1), buf_b, sem_b)
        return 0

    lax.fori_loop(0, BPW // 2, step, 0)

    drain(buf_a, sem_a)
    drain(buf_b, sem_b)

    pltpu.sync_copy(out_v, out_hbm.at[pl.ds(row0, BPW)])


@jax.jit
def _encode(ids2, table):
    mesh = plsc.VectorSubcoreMesh(core_axis_name="c", subcore_axis_name="s")
    run = pl.kernel(
        _body,
        out_type=jax.ShapeDtypeStruct((B, EMB), jnp.float32),
        mesh=mesh,
        scratch_types=[
            pltpu.VMEM((2 * BPW, HALF), jnp.int32),
            pltpu.VMEM((L, EMB), jnp.float32),
            pltpu.VMEM((L, EMB), jnp.float32),
            pltpu.VMEM((BPW, EMB), jnp.float32),
            pltpu.SemaphoreType.DMA,
            pltpu.SemaphoreType.DMA,
        ],
        compiler_params=pltpu.CompilerParams(use_tc_tiling_on_sc=False),
    )
    return run(ids2, table)


def kernel(input_ids, embedding_table):
    ids2 = input_ids.astype(jnp.int32).reshape(2 * B, HALF)
    out = _encode(ids2, embedding_table)
    return (out.reshape(B, 1, EMB),)
